# Initial kernel scaffold; baseline (speedup 1.0000x reference)
import jax
import jax.numpy as jnp
from jax.experimental import pallas as pl


def kernel(images, predictions):
    raise NotImplementedError("write your pallas kernel here")


# speedup vs baseline: 29.2761x; 29.2761x over previous
import functools

import numpy as np
import jax
import jax.numpy as jnp
from jax import lax
from jax.experimental import pallas as pl
from jax.experimental.pallas import tpu as pltpu
from jax.experimental.pallas import tpu_sc as plsc

_NUM_CLASSES = 80
_N_ANCHORS = 49104
_BATCH = 4
_TOPK = 200
_MAXDET = 100
_IOU_T = 0.5
_SCORE_T = 0.05
_BUF = 560
_SHRINK_AT = 496
_CHUNK = 16368
_I32_MIN = -2147483648
_I32_MAX = 2147483647
_NEG_INF = float("-inf")


def _anchors_np(image_h, image_w):
    aspect_ratios = [0.5, 1.0, 2.0]
    scales = [2.0 ** x for x in [0.0, 1.0 / 3.0, 2.0 / 3.0]]
    areas = [x ** 2 for x in [32.0, 64.0, 128.0, 256.0, 512.0]]
    all_anchors = []
    for level in range(3, 8):
        stride = 2 ** level
        fh = int(np.ceil(image_h / stride))
        fw = int(np.ceil(image_w / stride))
        dims = []
        area = areas[level - 3]
        for ratio in aspect_ratios:
            ah = np.sqrt(area / ratio)
            aw = area / ah
            for scale in scales:
                dims.append([scale * aw, scale * ah])
        dims = np.array(dims, dtype=np.float32)
        rx = (np.arange(fw, dtype=np.float32) + 0.5) * stride
        ry = (np.arange(fh, dtype=np.float32) + 0.5) * stride
        cx, cy = np.meshgrid(rx, ry)
        centers = np.stack([cx, cy], axis=-1)
        centers = np.tile(centers[:, :, None, :], [1, 1, 9, 1])
        d = np.tile(dims[None, None, :, :], [fh, fw, 1, 1])
        a = np.concatenate([centers, d], axis=-1).reshape(-1, 4)
        all_anchors.append(a.astype(np.float32))
    return np.concatenate(all_anchors, axis=0)


def _lane():
    return lax.iota(jnp.int32, 16)


def _splat_gather(ref, idx):
    return plsc.load_gather(ref, [jnp.zeros((16,), jnp.int32) + idx])


def _sload(ref, idx):
    return _splat_gather(ref, idx)[0]


def _sstore(ref, idx, val):
    iv = jnp.zeros((16,), jnp.int32) + idx
    vv = jnp.zeros((16,), val.dtype) + val
    plsc.store_scatter(ref, [iv], vv, mask=_lane() == 0)


def _popcnt(mask):
    return plsc.all_reduce_population_count(mask)


def _to_key(v):
    b = plsc.bitcast(v, jnp.int32)
    return jnp.where(b < 0, b ^ jnp.int32(0x7FFFFFFF), b)


def _from_key(k):
    b = jnp.where(k < 0, k ^ jnp.int32(0x7FFFFFFF), k)
    return plsc.bitcast(b, jnp.float32)


def _rank_thresh(kref, r):
    def bit_body(_, lohi):
        lo, hi = lohi
        d = hi - lo
        half = lax.shift_right_logical(d, 1)
        mid = lo + half + (d & 1)
        midv = jnp.full((16,), 0, jnp.int32) + mid
        acc = jnp.zeros((16,), jnp.int32)
        for i in range(_BUF // 16):
            k = kref[pl.ds(i * 16, 16)]
            acc = acc + _popcnt(k >= midv)
        ge = acc[0] >= r
        lo = jnp.where(ge, mid, lo)
        hi = jnp.where(ge, hi, mid - 1)
        return lo, hi
    lo, _ = lax.fori_loop(0, 32, bit_body,
                          (jnp.int32(_I32_MIN), jnp.int32(_I32_MAX)))
    return lo


def _count_gt(kref, t):
    tv = jnp.full((16,), 0, jnp.int32) + t
    acc = jnp.zeros((16,), jnp.int32)
    for i in range(_BUF // 16):
        acc = acc + _popcnt(kref[pl.ds(i * 16, 16)] > tv)
    return acc[0]


def _compact(skref, siref, dkref, diref, theta, tie_budget):
    zk = jnp.full((16,), _I32_MIN, jnp.int32)
    zi = jnp.zeros((16,), jnp.int32)
    for i in range(_BUF // 16):
        dkref[pl.ds(i * 16, 16)] = zk
        diref[pl.ds(i * 16, 16)] = zi
    tv = jnp.full((16,), 0, jnp.int32) + theta
    bv = jnp.full((16,), 0, jnp.int32) + tie_budget
    p = jnp.int32(0)
    ties = jnp.zeros((16,), jnp.int32)
    for i in range(_BUF // 16):
        k = skref[pl.ds(i * 16, 16)]
        iv = siref[pl.ds(i * 16, 16)]
        gt = k > tv
        eq = k == tv
        eqc = plsc.cumsum(eq.astype(jnp.int32))
        keep = jnp.logical_or(gt, jnp.logical_and(eq, (ties + eqc) <= bv))
        plsc.store_compressed(dkref.at[pl.ds(p, 16)], k, mask=keep)
        plsc.store_compressed(diref.at[pl.ds(p, 16)], iv, mask=keep)
        p = p + _popcnt(keep)[0]
        ties = ties + _popcnt(eq)
    return p


def _sc_body(logits_hbm, tab_hbm,
             ob_hbm, os_hbm, oc_hbm, onv_hbm,
             chunk, bkA, biA, bkB, biB, idxg0, idxg1, prows,
             x1r, y1r, x2r, y2r, arr, scr, st_s, st_b,
             msc, mbx, mk, hk, hp, ob_st, os_st, oc_st, onv_st,
             ssc, sbx):
    cid = lax.axis_index("c")
    sid = lax.axis_index("s")
    lanes = _lane()
    zk16 = jnp.full((16,), _I32_MIN, jnp.int32)

    def one_problem(p, carry0):
        lb = p // 5
        j = p - 5 * lb
        b = 2 * cid + lb
        c = sid * 5 + j
        r = b * _NUM_CLASSES + c

        for i in range(_BUF // 16):
            bkA[pl.ds(i * 16, 16)] = zk16

        def chunk_body(ch, carry):
            P, theta = carry
            pltpu.sync_copy(
                logits_hbm.at[pl.ds(r * _N_ANCHORS + ch * _CHUNK, _CHUNK)],
                chunk)
            cbase = ch * _CHUNK

            def scan_body(t, carry2):
                P2, th2 = carry2
                thv = jnp.full((16,), 0, jnp.int32) + th2
                for u in range(3):
                    off = t * 48 + u * 16
                    v = chunk[pl.ds(off, 16)]
                    k = _to_key(v)
                    m = k > thv
                    iv = lanes + (cbase + off)
                    plsc.store_compressed(bkA.at[pl.ds(P2, 16)], k, mask=m)
                    plsc.store_compressed(biA.at[pl.ds(P2, 16)], iv, mask=m)
                    P2 = P2 + _popcnt(m)[0]

                def do_shrink(op):
                    Ps, _ths = op
                    t200 = _rank_thresh(bkA, _TOPK)
                    newP = _compact(bkA, biA, bkB, biB, t200,
                                    jnp.int32(_TOPK))
                    for i in range(_BUF // 16):
                        bkA[pl.ds(i * 16, 16)] = bkB[pl.ds(i * 16, 16)]
                        biA[pl.ds(i * 16, 16)] = biB[pl.ds(i * 16, 16)]
                    return newP, t200

                P2, th2 = lax.cond(P2 >= _SHRINK_AT, do_shrink,
                                   lambda op: op, (P2, th2))
                return P2, th2

            P, theta = lax.fori_loop(0, _CHUNK // 48, scan_body, (P, theta))
            return P, theta

        lax.fori_loop(0, _N_ANCHORS // _CHUNK, chunk_body,
                      (jnp.int32(0), jnp.int32(_I32_MIN)))

        tstar = _rank_thresh(bkA, _TOPK)
        m_gt = _count_gt(bkA, tstar)
        _compact(bkA, biA, bkB, biB, tstar, _TOPK - m_gt)

        basep = b * _N_ANCHORS
        bpv = jnp.full((16,), 0, jnp.int32) + basep
        for i in range(7):
            idxg0[pl.ds(i * 16, 16)] = biB[pl.ds(i * 16, 16)] + bpv
        for i in range(7):
            idxg1[pl.ds(i * 16, 16)] = biB[pl.ds((i + 7) * 16, 16)] + bpv
        pltpu.sync_copy(tab_hbm.at[idxg0], prows.at[pl.ds(0, 112), :])
        pltpu.sync_copy(tab_hbm.at[idxg1], prows.at[pl.ds(112, 112), :])

        for i in range(13):
            rows = lanes + (i * 16)
            c0 = jnp.zeros((16,), jnp.int32)
            px = plsc.load_gather(prows, [rows, c0])
            py = plsc.load_gather(prows, [rows, c0 + 1])
            pw = plsc.load_gather(prows, [rows, c0 + 2])
            ph = plsc.load_gather(prows, [rows, c0 + 3])
            ax = plsc.load_gather(prows, [rows, c0 + 4])
            ay = plsc.load_gather(prows, [rows, c0 + 5])
            aw = plsc.load_gather(prows, [rows, c0 + 6])
            ah = plsc.load_gather(prows, [rows, c0 + 7])
            x = px * jnp.float32(0.1) * aw + ax
            y = py * jnp.float32(0.1) * ah + ay
            w = jnp.exp(pw * jnp.float32(0.2)) * aw
            h = jnp.exp(ph * jnp.float32(0.2)) * ah
            x1 = x - w * jnp.float32(0.5)
            y1 = y - h * jnp.float32(0.5)
            x2 = x + w * jnp.float32(0.5)
            y2 = y + h * jnp.float32(0.5)
            x1r[pl.ds(i * 16, 16)] = x1
            y1r[pl.ds(i * 16, 16)] = y1
            x2r[pl.ds(i * 16, 16)] = x2
            y2r[pl.ds(i * 16, 16)] = y2
            arr[pl.ds(i * 16, 16)] = (x2 - x1) * (y2 - y1)
            lg = _from_key(bkB[pl.ds(i * 16, 16)])
            s0 = jnp.float32(1.0) / (jnp.float32(1.0) + jnp.exp(-lg))
            ok = s0 >= jnp.float32(_SCORE_T)
            if i == 12:
                ok = jnp.logical_and(ok, lanes < 8)
            scr[pl.ds(i * 16, 16)] = jnp.where(ok, s0, _NEG_INF)

        zf = jnp.zeros((16,), jnp.float32)
        for i in range(7):
            st_s[pl.ds(i * 16, 16)] = zf
        for i in range(28):
            st_b[pl.ds(i * 16, 16)] = zf

        def nms_cond(carry):
            i, done = carry
            return jnp.logical_and(i < _MAXDET, jnp.logical_not(done))

        def nms_body(carry):
            i, _ = carry
            mval = jnp.full((16,), _NEG_INF, jnp.float32)
            midx = jnp.zeros((16,), jnp.int32)
            for q in range(13):
                v = scr[pl.ds(q * 16, 16)]
                take = v > mval
                mval = jnp.where(take, v, mval)
                midx = jnp.where(take, lanes + (q * 16), midx)
            best = jnp.max(mval)
            valid = best > jnp.float32(0.0)
            cand = jnp.where(mval == best, midx, jnp.int32(1 << 30))
            bidx = jnp.min(cand)

            @pl.when(valid)
            def _():
                bx1 = _splat_gather(x1r, bidx)
                by1 = _splat_gather(y1r, bidx)
                bx2 = _splat_gather(x2r, bidx)
                by2 = _splat_gather(y2r, bidx)
                ba = _splat_gather(arr, bidx)
                for q in range(13):
                    xx1 = jnp.maximum(bx1, x1r[pl.ds(q * 16, 16)])
                    yy1 = jnp.maximum(by1, y1r[pl.ds(q * 16, 16)])
                    xx2 = jnp.minimum(bx2, x2r[pl.ds(q * 16, 16)])
                    yy2 = jnp.minimum(by2, y2r[pl.ds(q * 16, 16)])
                    inter = (jnp.maximum(xx2 - xx1, jnp.float32(0.0)) *
                             jnp.maximum(yy2 - yy1, jnp.float32(0.0)))
                    a2 = arr[pl.ds(q * 16, 16)]
                    den = jnp.maximum(ba + a2 - inter, jnp.float32(1e-8))
                    iou = inter / den
                    sv = scr[pl.ds(q * 16, 16)]
                    scr[pl.ds(q * 16, 16)] = jnp.where(
                        iou > jnp.float32(_IOU_T), _NEG_INF, sv)
                _sstore(st_s, i, best)
                boxv = jnp.where(lanes == 0, bx1,
                                 jnp.where(lanes == 1, by1,
                                           jnp.where(lanes == 2, bx2, by2)))
                plsc.store_scatter(st_b, [4 * i + lanes], boxv,
                                   mask=lanes < 4)

            return i + 1, jnp.logical_not(valid)

        lax.while_loop(nms_cond, nms_body, (jnp.int32(0), False))

        pltpu.sync_copy(st_s.at[pl.ds(0, 104)],
                        ssc.at[pl.ds(lb * 8320 + c * 104, 104)])
        pltpu.sync_copy(st_b.at[pl.ds(0, 416)],
                        sbx.at[pl.ds(lb * 33280 + c * 416, 416)])
        return carry0

    lax.fori_loop(0, 10, one_problem, 0)
    plsc.subcore_barrier()

    @pl.when(sid < 2)
    def _():
        lb = sid
        bsel = 2 * cid + lb
        pltpu.sync_copy(ssc.at[pl.ds(lb * 8320, 8320)], msc)
        pltpu.sync_copy(sbx.at[pl.ds(lb * 33280, 33280)], mbx)

        def key_body(t, kcarry):
            for u in range(4):
                off = t * 64 + u * 16
                s = msc[pl.ds(off, 16)]
                bbits = plsc.bitcast(s, jnp.int32)
                mk[pl.ds(off, 16)] = jnp.where(s > jnp.float32(0.0),
                                               bbits, zk16)
            return kcarry

        lax.fori_loop(0, 8320 // 64, key_body, 0)
        for q in range(5):
            cvec = lanes + (q * 16)
            hk[pl.ds(q * 16, 16)] = plsc.load_gather(mk, [cvec * 104])
            hp[pl.ds(q * 16, 16)] = jnp.zeros((16,), jnp.int32)

        zf = jnp.zeros((16,), jnp.float32)
        for i in range(32):
            ob_st[pl.ds(i * 16, 16)] = zf
        for i in range(8):
            os_st[pl.ds(i * 16, 16)] = zf
            oc_st[pl.ds(i * 16, 16)] = zf
        onv_st[pl.ds(0, 16)] = jnp.zeros((16,), jnp.int32)

        def mg_cond(carry):
            jj, done, nv = carry
            return jnp.logical_and(jj < _MAXDET, jnp.logical_not(done))

        def mg_body(carry):
            jj, _, nv = carry
            mval = jnp.full((16,), _I32_MIN, jnp.int32)
            midx = jnp.zeros((16,), jnp.int32)
            for q in range(5):
                v = hk[pl.ds(q * 16, 16)]
                take = v > mval
                mval = jnp.where(take, v, mval)
                midx = jnp.where(take, lanes + (q * 16), midx)
            bestk = jnp.max(mval)
            valid = bestk > _I32_MIN
            cand = jnp.where(mval == bestk, midx, jnp.int32(1 << 30))
            bcls = jnp.min(cand)

            @pl.when(valid)
            def _():
                pos = _sload(hp, bcls)
                f = bcls * 104 + pos
                _sstore(os_st, jj, _sload(msc, f))
                _sstore(oc_st, jj, bcls.astype(jnp.float32))
                boxv = plsc.load_gather(mbx, [4 * f + lanes])
                plsc.store_scatter(ob_st, [4 * jj + lanes], boxv,
                                   mask=lanes < 4)
                _sstore(hp, bcls, pos + 1)
                _sstore(hk, bcls, _sload(mk, f + 1))

            return (jj + 1, jnp.logical_not(valid),
                    jnp.where(valid, nv + 1, nv))

        _, _, nv = lax.while_loop(mg_cond, mg_body,
                                  (jnp.int32(0), False, jnp.int32(0)))
        _sstore(onv_st, jnp.int32(0), nv)
        pltpu.sync_copy(ob_st, ob_hbm.at[pl.ds(bsel * 512, 512)])
        pltpu.sync_copy(os_st, os_hbm.at[pl.ds(bsel * 128, 128)])
        pltpu.sync_copy(oc_st, oc_hbm.at[pl.ds(bsel * 128, 128)])
        pltpu.sync_copy(onv_st.at[pl.ds(0, 8)],
                        onv_hbm.at[pl.ds(bsel * 8, 8)])


@functools.lru_cache(maxsize=2)
def _build_call(image_h, image_w):
    anch = _anchors_np(image_h, image_w)
    mesh = plsc.VectorSubcoreMesh(core_axis_name="c", subcore_axis_name="s")
    f32 = jnp.float32
    i32 = jnp.int32
    kern = pl.kernel(
        _sc_body,
        out_type=(jax.ShapeDtypeStruct((_BATCH * 512,), f32),
                  jax.ShapeDtypeStruct((_BATCH * 128,), f32),
                  jax.ShapeDtypeStruct((_BATCH * 128,), f32),
                  jax.ShapeDtypeStruct((_BATCH * 8,), i32)),
        mesh=mesh,
        compiler_params=pltpu.CompilerParams(needs_layout_passes=False,
                                             use_tc_tiling_on_sc=False),
        scratch_types=[
            pltpu.VMEM((_CHUNK,), f32),
            pltpu.VMEM((_BUF,), i32), pltpu.VMEM((_BUF,), i32),
            pltpu.VMEM((_BUF,), i32), pltpu.VMEM((_BUF,), i32),
            pltpu.VMEM((112,), i32), pltpu.VMEM((112,), i32),
            pltpu.VMEM((224, 16), f32),
            pltpu.VMEM((208,), f32), pltpu.VMEM((208,), f32),
            pltpu.VMEM((208,), f32), pltpu.VMEM((208,), f32),
            pltpu.VMEM((208,), f32), pltpu.VMEM((208,), f32),
            pltpu.VMEM((112,), f32), pltpu.VMEM((448,), f32),
            pltpu.VMEM((8320,), f32), pltpu.VMEM((33280,), f32),
            pltpu.VMEM((8320,), i32),
            pltpu.VMEM((80,), i32), pltpu.VMEM((80,), i32),
            pltpu.VMEM((512,), f32), pltpu.VMEM((128,), f32),
            pltpu.VMEM((128,), f32), pltpu.VMEM((16,), i32),
            pltpu.VMEM_SHARED((2 * 8320,), f32),
            pltpu.VMEM_SHARED((2 * 33280,), f32),
        ],
    )

    def run(predictions):
        lg = jnp.transpose(predictions[:, :, 4:], (0, 2, 1))
        lg = lg.reshape(_BATCH * _NUM_CLASSES * _N_ANCHORS)
        boxp = predictions[:, :, :4].reshape(_BATCH * _N_ANCHORS, 4)
        anch_t = jnp.broadcast_to(jnp.asarray(anch)[None],
                                  (_BATCH, _N_ANCHORS, 4))
        anch_t = anch_t.reshape(_BATCH * _N_ANCHORS, 4)
        pad = jnp.zeros((_BATCH * _N_ANCHORS, 8), jnp.float32)
        tab = jnp.concatenate([boxp, anch_t, pad], axis=1)
        ob, osc, ocl, onv = kern(lg, tab)
        return (ob.reshape(_BATCH, 128, 4)[:, :_MAXDET],
                osc.reshape(_BATCH, 128)[:, :_MAXDET],
                ocl.reshape(_BATCH, 128)[:, :_MAXDET],
                onv.reshape(_BATCH, 8)[:, 0])

    return run


def kernel(images, predictions):
    run = _build_call(images.shape[1], images.shape[2])
    return run(predictions)


# speedup vs baseline: 32.9576x; 1.1258x over previous
import functools

import numpy as np
import jax
import jax.numpy as jnp
from jax import lax
from jax.experimental import pallas as pl
from jax.experimental.pallas import tpu as pltpu
from jax.experimental.pallas import tpu_sc as plsc

_NUM_CLASSES = 80
_BATCH = 4
_N = 49104
_NPAD = 49152
_TOPK = 200
_MAXDET = 100
_IOU_T = 0.5
_SCORE_T = 0.05
_BUF = 672
_SHRINK_AT = 496
_CHUNK = 16368
_GRP = 11
_I32_MIN = -2147483648
_I32_MAX = 2147483647
_NEG_INF = float("-inf")


def _anchors_np(image_h, image_w):
    aspect_ratios = [0.5, 1.0, 2.0]
    scales = [2.0 ** x for x in [0.0, 1.0 / 3.0, 2.0 / 3.0]]
    areas = [x ** 2 for x in [32.0, 64.0, 128.0, 256.0, 512.0]]
    all_anchors = []
    for level in range(3, 8):
        stride = 2 ** level
        fh = int(np.ceil(image_h / stride))
        fw = int(np.ceil(image_w / stride))
        dims = []
        area = areas[level - 3]
        for ratio in aspect_ratios:
            ah = np.sqrt(area / ratio)
            aw = area / ah
            for scale in scales:
                dims.append([scale * aw, scale * ah])
        dims = np.array(dims, dtype=np.float32)
        rx = (np.arange(fw, dtype=np.float32) + 0.5) * stride
        ry = (np.arange(fh, dtype=np.float32) + 0.5) * stride
        cx, cy = np.meshgrid(rx, ry)
        centers = np.stack([cx, cy], axis=-1)
        centers = np.tile(centers[:, :, None, :], [1, 1, 9, 1])
        d = np.tile(dims[None, None, :, :], [fh, fw, 1, 1])
        a = np.concatenate([centers, d], axis=-1).reshape(-1, 4)
        all_anchors.append(a.astype(np.float32))
    return np.concatenate(all_anchors, axis=0)


def _lane():
    return lax.iota(jnp.int32, 16)


def _splat_gather(ref, idx):
    return plsc.load_gather(ref, [jnp.zeros((16,), jnp.int32) + idx])


def _sload(ref, idx):
    return _splat_gather(ref, idx)[0]


def _sstore(ref, idx, val):
    iv = jnp.zeros((16,), jnp.int32) + idx
    vv = jnp.zeros((16,), val.dtype) + val
    plsc.store_scatter(ref, [iv], vv, mask=_lane() == 0)


def _popcnt(mask):
    return plsc.all_reduce_population_count(mask)


def _to_key(v):
    b = plsc.bitcast(v, jnp.int32)
    return jnp.where(b < 0, b ^ jnp.int32(0x7FFFFFFF), b)


def _from_key(k):
    b = jnp.where(k < 0, k ^ jnp.int32(0x7FFFFFFF), k)
    return plsc.bitcast(b, jnp.float32)


def _fill_keys(iref, lgbuf, kref):
    for i in range(_BUF // 16):
        iv = iref[pl.ds(i * 16, 16)]
        v = plsc.load_gather(lgbuf, [iv])
        kref[pl.ds(i * 16, 16)] = _to_key(v)


def _rank_thresh(kref, r):
    def bit_body(_, lohi):
        lo, hi = lohi
        d = hi - lo
        half = lax.shift_right_logical(d, 1)
        mid = lo + half + (d & 1)
        midv = jnp.full((16,), 0, jnp.int32) + mid
        acc = jnp.zeros((16,), jnp.int32)
        for i in range(_BUF // 16):
            acc = acc + _popcnt(kref[pl.ds(i * 16, 16)] >= midv)
        ge = acc[0] >= r
        lo = jnp.where(ge, mid, lo)
        hi = jnp.where(ge, hi, mid - 1)
        return lo, hi
    lo, _ = lax.fori_loop(0, 32, bit_body,
                          (jnp.int32(_I32_MIN), jnp.int32(_I32_MAX)))
    return lo


def _count_gt(kref, t):
    tv = jnp.full((16,), 0, jnp.int32) + t
    acc = jnp.zeros((16,), jnp.int32)
    for i in range(_BUF // 16):
        acc = acc + _popcnt(kref[pl.ds(i * 16, 16)] > tv)
    return acc[0]


def _compact(kref, siref, diref, theta, tie_budget):
    pad = jnp.full((16,), _N, jnp.int32)
    for i in range(_BUF // 16):
        diref[pl.ds(i * 16, 16)] = pad
    tv = jnp.full((16,), 0, jnp.int32) + theta
    bv = jnp.full((16,), 0, jnp.int32) + tie_budget
    p = jnp.int32(0)
    ties = jnp.zeros((16,), jnp.int32)
    for i in range(_BUF // 16):
        k = kref[pl.ds(i * 16, 16)]
        iv = siref[pl.ds(i * 16, 16)]
        gt = k > tv
        eq = k == tv
        eqc = plsc.cumsum(eq.astype(jnp.int32))
        keep = jnp.logical_or(gt, jnp.logical_and(eq, (ties + eqc) <= bv))
        plsc.store_compressed(diref.at[pl.ds(p, 16)], iv, mask=keep)
        p = p + _popcnt(keep)[0]
        ties = ties + _popcnt(eq)
    return p


def _sc_body(logits_hbm, tab_hbm,
             ob_hbm, os_hbm, oc_hbm, onv_hbm,
             lgbuf, biA, biB, bkS, idxg0, idxg1, prows,
             x1r, y1r, x2r, y2r, arr, st_s, st_b,
             msc, mbx, mk, hk, hp, ob_st, os_st, oc_st, onv_st,
             dsem, ssc, sbx):
    cid = lax.axis_index("c")
    sid = lax.axis_index("s")
    lanes = _lane()
    zk16 = jnp.full((16,), _I32_MIN, jnp.int32)
    nanv = plsc.bitcast(jnp.full((16,), -1, jnp.int32), jnp.float32)

    def one_problem(p, carry0):
        lb = p // 5
        j = p - 5 * lb
        b = 2 * cid + lb
        c = sid * 5 + j
        r = b * _NUM_CLASSES + c

        for i in range(3):
            lgbuf[pl.ds(_N + i * 16, 16)] = nanv
        for i in range(_BUF // 16):
            biA[pl.ds(i * 16, 16)] = jnp.full((16,), _N, jnp.int32)

        for ch in range(3):
            pltpu.async_copy(
                logits_hbm.at[pl.ds(r * _N + ch * _CHUNK, _CHUNK)],
                lgbuf.at[pl.ds(ch * _CHUNK, _CHUNK)], dsem)

        def chunk_body(ch, carry):
            P, theta = carry
            pltpu.make_async_copy(
                logits_hbm.at[pl.ds(0, _CHUNK)],
                lgbuf.at[pl.ds(0, _CHUNK)], dsem).wait()
            cbase = ch * _CHUNK

            def scan_body(t, carry2):
                P2, th2 = carry2
                thv = jnp.full((16,), 0, jnp.int32) + th2
                for u in range(_GRP):
                    off = cbase + t * (_GRP * 16) + u * 16
                    k = _to_key(lgbuf[pl.ds(off, 16)])
                    m = k > thv
                    plsc.store_compressed(biA.at[pl.ds(P2, 16)],
                                          lanes + off, mask=m)
                    P2 = P2 + _popcnt(m)[0]

                def do_shrink(op):
                    _fill_keys(biA, lgbuf, bkS)
                    t200 = _rank_thresh(bkS, _TOPK)
                    newP = _compact(bkS, biA, biB, t200, jnp.int32(_TOPK))
                    for i in range(_BUF // 16):
                        biA[pl.ds(i * 16, 16)] = biB[pl.ds(i * 16, 16)]
                    return newP, t200

                P2, th2 = lax.cond(P2 >= _SHRINK_AT, do_shrink,
                                   lambda op: op, (P2, th2))
                return P2, th2

            return lax.fori_loop(0, _CHUNK // (_GRP * 16), scan_body,
                                 (P, theta))

        lax.fori_loop(0, 3, chunk_body,
                      (jnp.int32(0), jnp.int32(_I32_MIN)))

        _fill_keys(biA, lgbuf, bkS)
        tstar = _rank_thresh(bkS, _TOPK)
        m_gt = _count_gt(bkS, tstar)
        _compact(bkS, biA, biB, tstar, _TOPK - m_gt)

        bpv = jnp.full((16,), 0, jnp.int32) + b * _N
        for i in range(7):
            idxg0[pl.ds(i * 16, 16)] = biB[pl.ds(i * 16, 16)] + bpv
        for i in range(7):
            idxg1[pl.ds(i * 16, 16)] = biB[pl.ds((i + 7) * 16, 16)] + bpv
        pltpu.sync_copy(tab_hbm.at[idxg0], prows.at[pl.ds(0, 112), :])
        pltpu.sync_copy(tab_hbm.at[idxg1], prows.at[pl.ds(112, 112), :])

        svecs = []
        for i in range(13):
            rows = lanes + (i * 16)
            c0 = jnp.zeros((16,), jnp.int32)
            px = plsc.load_gather(prows, [rows, c0])
            py = plsc.load_gather(prows, [rows, c0 + 1])
            pw = plsc.load_gather(prows, [rows, c0 + 2])
            ph = plsc.load_gather(prows, [rows, c0 + 3])
            ax = plsc.load_gather(prows, [rows, c0 + 4])
            ay = plsc.load_gather(prows, [rows, c0 + 5])
            aw = plsc.load_gather(prows, [rows, c0 + 6])
            ah = plsc.load_gather(prows, [rows, c0 + 7])
            x = px * jnp.float32(0.1) * aw + ax
            y = py * jnp.float32(0.1) * ah + ay
            w = jnp.exp(pw * jnp.float32(0.2)) * aw
            h = jnp.exp(ph * jnp.float32(0.2)) * ah
            x1 = x - w * jnp.float32(0.5)
            y1 = y - h * jnp.float32(0.5)
            x2 = x + w * jnp.float32(0.5)
            y2 = y + h * jnp.float32(0.5)
            x1r[pl.ds(i * 16, 16)] = x1
            y1r[pl.ds(i * 16, 16)] = y1
            x2r[pl.ds(i * 16, 16)] = x2
            y2r[pl.ds(i * 16, 16)] = y2
            arr[pl.ds(i * 16, 16)] = (x2 - x1) * (y2 - y1)
            iv = biB[pl.ds(i * 16, 16)]
            lg = plsc.load_gather(lgbuf, [iv])
            s0 = jnp.float32(1.0) / (jnp.float32(1.0) + jnp.exp(-lg))
            ok = s0 >= jnp.float32(_SCORE_T)
            if i == 12:
                ok = jnp.logical_and(ok, lanes < 8)
            svecs.append(jnp.where(ok, s0, _NEG_INF))

        zf = jnp.zeros((16,), jnp.float32)
        for i in range(7):
            st_s[pl.ds(i * 16, 16)] = zf
        for i in range(28):
            st_b[pl.ds(i * 16, 16)] = zf

        def nms_cond(carry):
            i, done = carry[0], carry[1]
            return jnp.logical_and(i < _MAXDET, jnp.logical_not(done))

        def nms_body(carry):
            i = carry[0]
            s = list(carry[2])
            mval = jnp.full((16,), _NEG_INF, jnp.float32)
            midx = jnp.zeros((16,), jnp.int32)
            for q in range(13):
                take = s[q] > mval
                mval = jnp.where(take, s[q], mval)
                midx = jnp.where(take, lanes + (q * 16), midx)
            best = jnp.max(mval)
            valid = best > jnp.float32(0.0)
            cand = jnp.where(mval == best, midx, jnp.int32(1 << 30))
            bidx = jnp.where(valid, jnp.min(cand), jnp.int32(0))

            bx1 = _splat_gather(x1r, bidx)
            by1 = _splat_gather(y1r, bidx)
            bx2 = _splat_gather(x2r, bidx)
            by2 = _splat_gather(y2r, bidx)
            ba = _splat_gather(arr, bidx)
            for q in range(13):
                xx1 = jnp.maximum(bx1, x1r[pl.ds(q * 16, 16)])
                yy1 = jnp.maximum(by1, y1r[pl.ds(q * 16, 16)])
                xx2 = jnp.minimum(bx2, x2r[pl.ds(q * 16, 16)])
                yy2 = jnp.minimum(by2, y2r[pl.ds(q * 16, 16)])
                inter = (jnp.maximum(xx2 - xx1, jnp.float32(0.0)) *
                         jnp.maximum(yy2 - yy1, jnp.float32(0.0)))
                a2 = arr[pl.ds(q * 16, 16)]
                den = jnp.maximum(ba + a2 - inter, jnp.float32(1e-8))
                iou = inter / den
                supp = jnp.logical_and(iou > jnp.float32(_IOU_T), valid)
                s[q] = jnp.where(supp, _NEG_INF, s[q])

            @pl.when(valid)
            def _():
                _sstore(st_s, i, best)
                boxv = jnp.where(lanes == 0, bx1,
                                 jnp.where(lanes == 1, by1,
                                           jnp.where(lanes == 2, bx2, by2)))
                plsc.store_scatter(st_b, [4 * i + lanes], boxv,
                                   mask=lanes < 4)

            return (i + 1, jnp.logical_not(valid), tuple(s))

        lax.while_loop(nms_cond, nms_body,
                       (jnp.int32(0), False, tuple(svecs)))

        pltpu.sync_copy(st_s.at[pl.ds(0, 104)],
                        ssc.at[pl.ds(lb * 8320 + c * 104, 104)])
        pltpu.sync_copy(st_b.at[pl.ds(0, 416)],
                        sbx.at[pl.ds(lb * 33280 + c * 416, 416)])
        return carry0

    lax.fori_loop(0, 10, one_problem, 0)
    plsc.subcore_barrier()

    @pl.when(sid < 2)
    def _():
        lb = sid
        bsel = 2 * cid + lb
        pltpu.sync_copy(ssc.at[pl.ds(lb * 8320, 8320)], msc)
        pltpu.sync_copy(sbx.at[pl.ds(lb * 33280, 33280)], mbx)

        def key_body(t, kcarry):
            for u in range(4):
                off = t * 64 + u * 16
                s = msc[pl.ds(off, 16)]
                bbits = plsc.bitcast(s, jnp.int32)
                mk[pl.ds(off, 16)] = jnp.where(s > jnp.float32(0.0),
                                               bbits, zk16)
            return kcarry

        lax.fori_loop(0, 8320 // 64, key_body, 0)
        for q in range(5):
            cvec = lanes + (q * 16)
            hk[pl.ds(q * 16, 16)] = plsc.load_gather(mk, [cvec * 104])
            hp[pl.ds(q * 16, 16)] = jnp.zeros((16,), jnp.int32)

        zf = jnp.zeros((16,), jnp.float32)
        for i in range(32):
            ob_st[pl.ds(i * 16, 16)] = zf
        for i in range(8):
            os_st[pl.ds(i * 16, 16)] = zf
            oc_st[pl.ds(i * 16, 16)] = zf
        onv_st[pl.ds(0, 16)] = jnp.zeros((16,), jnp.int32)

        def mg_cond(carry):
            jj, done, nv = carry
            return jnp.logical_and(jj < _MAXDET, jnp.logical_not(done))

        def mg_body(carry):
            jj, _, nv = carry
            mval = jnp.full((16,), _I32_MIN, jnp.int32)
            midx = jnp.zeros((16,), jnp.int32)
            for q in range(5):
                v = hk[pl.ds(q * 16, 16)]
                take = v > mval
                mval = jnp.where(take, v, mval)
                midx = jnp.where(take, lanes + (q * 16), midx)
            bestk = jnp.max(mval)
            valid = bestk > _I32_MIN
            cand = jnp.where(mval == bestk, midx, jnp.int32(1 << 30))
            bcls = jnp.where(valid, jnp.min(cand), jnp.int32(0))

            @pl.when(valid)
            def _():
                pos = _sload(hp, bcls)
                f = bcls * 104 + pos
                _sstore(os_st, jj, _sload(msc, f))
                _sstore(oc_st, jj, bcls.astype(jnp.float32))
                boxv = plsc.load_gather(mbx, [4 * f + lanes])
                plsc.store_scatter(ob_st, [4 * jj + lanes], boxv,
                                   mask=lanes < 4)
                _sstore(hp, bcls, pos + 1)
                _sstore(hk, bcls, _sload(mk, f + 1))

            return (jj + 1, jnp.logical_not(valid),
                    jnp.where(valid, nv + 1, nv))

        _, _, nv = lax.while_loop(mg_cond, mg_body,
                                  (jnp.int32(0), False, jnp.int32(0)))
        _sstore(onv_st, jnp.int32(0), nv)
        pltpu.sync_copy(ob_st, ob_hbm.at[pl.ds(bsel * 512, 512)])
        pltpu.sync_copy(os_st, os_hbm.at[pl.ds(bsel * 128, 128)])
        pltpu.sync_copy(oc_st, oc_hbm.at[pl.ds(bsel * 128, 128)])
        pltpu.sync_copy(onv_st.at[pl.ds(0, 8)],
                        onv_hbm.at[pl.ds(bsel * 8, 8)])


@functools.lru_cache(maxsize=2)
def _build_call(image_h, image_w):
    anch = _anchors_np(image_h, image_w)
    mesh = plsc.VectorSubcoreMesh(core_axis_name="c", subcore_axis_name="s")
    f32 = jnp.float32
    i32 = jnp.int32
    kern = pl.kernel(
        _sc_body,
        out_type=(jax.ShapeDtypeStruct((_BATCH * 512,), f32),
                  jax.ShapeDtypeStruct((_BATCH * 128,), f32),
                  jax.ShapeDtypeStruct((_BATCH * 128,), f32),
                  jax.ShapeDtypeStruct((_BATCH * 8,), i32)),
        mesh=mesh,
        compiler_params=pltpu.CompilerParams(needs_layout_passes=False,
                                             use_tc_tiling_on_sc=False),
        scratch_types=[
            pltpu.VMEM((_NPAD,), f32),
            pltpu.VMEM((_BUF,), i32), pltpu.VMEM((_BUF,), i32),
            pltpu.VMEM((_BUF,), i32),
            pltpu.VMEM((112,), i32), pltpu.VMEM((112,), i32),
            pltpu.VMEM((224, 16), f32),
            pltpu.VMEM((208,), f32), pltpu.VMEM((208,), f32),
            pltpu.VMEM((208,), f32), pltpu.VMEM((208,), f32),
            pltpu.VMEM((208,), f32),
            pltpu.VMEM((112,), f32), pltpu.VMEM((448,), f32),
            pltpu.VMEM((8320,), f32), pltpu.VMEM((33280,), f32),
            pltpu.VMEM((8320,), i32),
            pltpu.VMEM((80,), i32), pltpu.VMEM((80,), i32),
            pltpu.VMEM((512,), f32), pltpu.VMEM((128,), f32),
            pltpu.VMEM((128,), f32), pltpu.VMEM((16,), i32),
            pltpu.SemaphoreType.DMA,
            pltpu.VMEM_SHARED((2 * 8320,), f32),
            pltpu.VMEM_SHARED((2 * 33280,), f32),
        ],
    )

    def run(predictions):
        lg = jnp.transpose(predictions[:, :, 4:], (0, 2, 1))
        lg = lg.reshape(_BATCH * _NUM_CLASSES * _N)
        boxp = predictions[:, :, :4].reshape(_BATCH * _N, 4)
        anch_t = jnp.broadcast_to(jnp.asarray(anch)[None],
                                  (_BATCH, _N, 4))
        anch_t = anch_t.reshape(_BATCH * _N, 4)
        pad = jnp.zeros((_BATCH * _N, 8), jnp.float32)
        tab = jnp.concatenate([boxp, anch_t, pad], axis=1)
        ob, osc, ocl, onv = kern(lg, tab)
        return (ob.reshape(_BATCH, 128, 4)[:, :_MAXDET],
                osc.reshape(_BATCH, 128)[:, :_MAXDET],
                ocl.reshape(_BATCH, 128)[:, :_MAXDET],
                onv.reshape(_BATCH, 8)[:, 0])

    return run


def kernel(images, predictions):
    run = _build_call(images.shape[1], images.shape[2])
    return run(predictions)


# speedup vs baseline: 32.9707x; 1.0004x over previous
import functools

import numpy as np
import jax
import jax.numpy as jnp
from jax import lax
from jax.experimental import pallas as pl
from jax.experimental.pallas import tpu as pltpu
from jax.experimental.pallas import tpu_sc as plsc

_NUM_CLASSES = 80
_BATCH = 4
_N = 49104
_NPAD = 49152
_TOPK = 200
_MAXDET = 100
_IOU_T = 0.5
_SCORE_T = 0.05
_BUF = 672
_SHRINK_AT = 496
_CHUNK = 16368
_GRP = 11
_I32_MIN = -2147483648
_I32_MAX = 2147483647
_NEG_INF = float("-inf")


def _anchors_np(image_h, image_w):
    aspect_ratios = [0.5, 1.0, 2.0]
    scales = [2.0 ** x for x in [0.0, 1.0 / 3.0, 2.0 / 3.0]]
    areas = [x ** 2 for x in [32.0, 64.0, 128.0, 256.0, 512.0]]
    all_anchors = []
    for level in range(3, 8):
        stride = 2 ** level
        fh = int(np.ceil(image_h / stride))
        fw = int(np.ceil(image_w / stride))
        dims = []
        area = areas[level - 3]
        for ratio in aspect_ratios:
            ah = np.sqrt(area / ratio)
            aw = area / ah
            for scale in scales:
                dims.append([scale * aw, scale * ah])
        dims = np.array(dims, dtype=np.float32)
        rx = (np.arange(fw, dtype=np.float32) + 0.5) * stride
        ry = (np.arange(fh, dtype=np.float32) + 0.5) * stride
        cx, cy = np.meshgrid(rx, ry)
        centers = np.stack([cx, cy], axis=-1)
        centers = np.tile(centers[:, :, None, :], [1, 1, 9, 1])
        d = np.tile(dims[None, None, :, :], [fh, fw, 1, 1])
        a = np.concatenate([centers, d], axis=-1).reshape(-1, 4)
        all_anchors.append(a.astype(np.float32))
    return np.concatenate(all_anchors, axis=0)


def _lane():
    return lax.iota(jnp.int32, 16)


def _splat_gather(ref, idx):
    return plsc.load_gather(ref, [jnp.zeros((16,), jnp.int32) + idx])


def _sload(ref, idx):
    return _splat_gather(ref, idx)[0]


def _sstore(ref, idx, val):
    iv = jnp.zeros((16,), jnp.int32) + idx
    vv = jnp.zeros((16,), val.dtype) + val
    plsc.store_scatter(ref, [iv], vv, mask=_lane() == 0)


def _popcnt(mask):
    return plsc.all_reduce_population_count(mask)


def _to_key(v):
    b = plsc.bitcast(v, jnp.int32)
    return jnp.where(b < 0, b ^ jnp.int32(0x7FFFFFFF), b)


def _from_key(k):
    b = jnp.where(k < 0, k ^ jnp.int32(0x7FFFFFFF), k)
    return plsc.bitcast(b, jnp.float32)


def _fill_keys(iref, lgbuf, kref):
    for i in range(_BUF // 16):
        iv = iref[pl.ds(i * 16, 16)]
        v = plsc.load_gather(lgbuf, [iv])
        kref[pl.ds(i * 16, 16)] = _to_key(v)


def _rank_thresh(kref, r):
    def bit_body(_, lohi):
        lo, hi = lohi
        d = hi - lo
        half = lax.shift_right_logical(d, 1)
        mid = lo + half + (d & 1)
        midv = jnp.full((16,), 0, jnp.int32) + mid
        acc = jnp.zeros((16,), jnp.int32)
        for i in range(_BUF // 16):
            acc = acc + _popcnt(kref[pl.ds(i * 16, 16)] >= midv)
        ge = acc[0] >= r
        lo = jnp.where(ge, mid, lo)
        hi = jnp.where(ge, hi, mid - 1)
        return lo, hi
    lo, _ = lax.fori_loop(0, 32, bit_body,
                          (jnp.int32(_I32_MIN), jnp.int32(_I32_MAX)))
    return lo


def _count_gt(kref, t):
    tv = jnp.full((16,), 0, jnp.int32) + t
    acc = jnp.zeros((16,), jnp.int32)
    for i in range(_BUF // 16):
        acc = acc + _popcnt(kref[pl.ds(i * 16, 16)] > tv)
    return acc[0]


def _compact(kref, siref, diref, theta, tie_budget):
    pad = jnp.full((16,), _N, jnp.int32)
    for i in range(_BUF // 16):
        diref[pl.ds(i * 16, 16)] = pad
    tv = jnp.full((16,), 0, jnp.int32) + theta
    bv = jnp.full((16,), 0, jnp.int32) + tie_budget
    p = jnp.int32(0)
    ties = jnp.zeros((16,), jnp.int32)
    for i in range(_BUF // 16):
        k = kref[pl.ds(i * 16, 16)]
        iv = siref[pl.ds(i * 16, 16)]
        gt = k > tv
        eq = k == tv
        eqc = plsc.cumsum(eq.astype(jnp.int32))
        keep = jnp.logical_or(gt, jnp.logical_and(eq, (ties + eqc) <= bv))
        plsc.store_compressed(diref.at[pl.ds(p, 16)], iv, mask=keep)
        p = p + _popcnt(keep)[0]
        ties = ties + _popcnt(eq)
    return p


def _sc_body(logits_hbm, tab_hbm,
             ob_hbm, os_hbm, oc_hbm, onv_hbm,
             lgbuf, biA, biB, bkS, idxg0, idxg1, prows,
             x1r, y1r, x2r, y2r, arr, st_s, st_b,
             msc, mbx, mk, hk, hp, ob_st, os_st, oc_st, onv_st,
             dsem, ssc, sbx):
    cid = lax.axis_index("c")
    sid = lax.axis_index("s")
    lanes = _lane()
    zk16 = jnp.full((16,), _I32_MIN, jnp.int32)
    nanv = plsc.bitcast(jnp.full((16,), -1, jnp.int32), jnp.float32)

    def one_problem(p, carry0):
        lb = p // 5
        j = p - 5 * lb
        b = 2 * cid + lb
        c = sid * 5 + j
        r = b * _NUM_CLASSES + c

        for i in range(3):
            lgbuf[pl.ds(_N + i * 16, 16)] = nanv
        for i in range(_BUF // 16):
            biA[pl.ds(i * 16, 16)] = jnp.full((16,), _N, jnp.int32)

        for ch in range(3):
            pltpu.async_copy(
                logits_hbm.at[pl.ds(r * _N + ch * _CHUNK, _CHUNK)],
                lgbuf.at[pl.ds(ch * _CHUNK, _CHUNK)], dsem)

        def chunk_body(ch, carry):
            P, theta = carry
            pltpu.make_async_copy(
                logits_hbm.at[pl.ds(0, _CHUNK)],
                lgbuf.at[pl.ds(0, _CHUNK)], dsem).wait()
            cbase = ch * _CHUNK

            def scan_body(t, carry2):
                P2, th2 = carry2
                thv = jnp.full((16,), 0, jnp.int32) + th2
                for u in range(_GRP):
                    off = cbase + t * (_GRP * 16) + u * 16
                    k = _to_key(lgbuf[pl.ds(off, 16)])
                    m = k > thv
                    plsc.store_compressed(biA.at[pl.ds(P2, 16)],
                                          lanes + off, mask=m)
                    P2 = P2 + _popcnt(m)[0]

                def do_shrink(op):
                    _fill_keys(biA, lgbuf, bkS)
                    t200 = _rank_thresh(bkS, _TOPK)
                    newP = _compact(bkS, biA, biB, t200, jnp.int32(_TOPK))
                    for i in range(_BUF // 16):
                        biA[pl.ds(i * 16, 16)] = biB[pl.ds(i * 16, 16)]
                    return newP, t200

                P2, th2 = lax.cond(P2 >= _SHRINK_AT, do_shrink,
                                   lambda op: op, (P2, th2))
                return P2, th2

            return lax.fori_loop(0, _CHUNK // (_GRP * 16), scan_body,
                                 (P, theta))

        with jax.named_scope("scan"):
            lax.fori_loop(0, 3, chunk_body,
                          (jnp.int32(0), jnp.int32(_I32_MIN)))

        with jax.named_scope("select"):
            _fill_keys(biA, lgbuf, bkS)
            tstar = _rank_thresh(bkS, _TOPK)
            m_gt = _count_gt(bkS, tstar)
            _compact(bkS, biA, biB, tstar, _TOPK - m_gt)

        bpv = jnp.full((16,), 0, jnp.int32) + b * _N
        for i in range(7):
            idxg0[pl.ds(i * 16, 16)] = biB[pl.ds(i * 16, 16)] + bpv
        for i in range(7):
            idxg1[pl.ds(i * 16, 16)] = biB[pl.ds((i + 7) * 16, 16)] + bpv
        with jax.named_scope("gather"):
            pltpu.sync_copy(tab_hbm.at[idxg0], prows.at[pl.ds(0, 112), :])
            pltpu.sync_copy(tab_hbm.at[idxg1],
                            prows.at[pl.ds(112, 112), :])

        svecs = []
        for i in range(13):
            rows = lanes + (i * 16)
            c0 = jnp.zeros((16,), jnp.int32)
            px = plsc.load_gather(prows, [rows, c0])
            py = plsc.load_gather(prows, [rows, c0 + 1])
            pw = plsc.load_gather(prows, [rows, c0 + 2])
            ph = plsc.load_gather(prows, [rows, c0 + 3])
            ax = plsc.load_gather(prows, [rows, c0 + 4])
            ay = plsc.load_gather(prows, [rows, c0 + 5])
            aw = plsc.load_gather(prows, [rows, c0 + 6])
            ah = plsc.load_gather(prows, [rows, c0 + 7])
            x = px * jnp.float32(0.1) * aw + ax
            y = py * jnp.float32(0.1) * ah + ay
            w = jnp.exp(pw * jnp.float32(0.2)) * aw
            h = jnp.exp(ph * jnp.float32(0.2)) * ah
            x1 = x - w * jnp.float32(0.5)
            y1 = y - h * jnp.float32(0.5)
            x2 = x + w * jnp.float32(0.5)
            y2 = y + h * jnp.float32(0.5)
            x1r[pl.ds(i * 16, 16)] = x1
            y1r[pl.ds(i * 16, 16)] = y1
            x2r[pl.ds(i * 16, 16)] = x2
            y2r[pl.ds(i * 16, 16)] = y2
            arr[pl.ds(i * 16, 16)] = (x2 - x1) * (y2 - y1)
            iv = biB[pl.ds(i * 16, 16)]
            lg = plsc.load_gather(lgbuf, [iv])
            s0 = jnp.float32(1.0) / (jnp.float32(1.0) + jnp.exp(-lg))
            ok = s0 >= jnp.float32(_SCORE_T)
            if i == 12:
                ok = jnp.logical_and(ok, lanes < 8)
            svecs.append(jnp.where(ok, s0, _NEG_INF))

        zf = jnp.zeros((16,), jnp.float32)
        for i in range(7):
            st_s[pl.ds(i * 16, 16)] = zf
        for i in range(28):
            st_b[pl.ds(i * 16, 16)] = zf

        def nms_cond(carry):
            i, done = carry[0], carry[1]
            return jnp.logical_and(i < _MAXDET, jnp.logical_not(done))

        def nms_body(carry):
            i = carry[0]
            s = list(carry[2])
            mval = jnp.full((16,), _NEG_INF, jnp.float32)
            midx = jnp.zeros((16,), jnp.int32)
            for q in range(13):
                take = s[q] > mval
                mval = jnp.where(take, s[q], mval)
                midx = jnp.where(take, lanes + (q * 16), midx)
            best = jnp.max(mval)
            valid = best > jnp.float32(0.0)
            cand = jnp.where(mval == best, midx, jnp.int32(1 << 30))
            bidx = jnp.where(valid, jnp.min(cand), jnp.int32(0))

            bx1 = _splat_gather(x1r, bidx)
            by1 = _splat_gather(y1r, bidx)
            bx2 = _splat_gather(x2r, bidx)
            by2 = _splat_gather(y2r, bidx)
            ba = _splat_gather(arr, bidx)
            for q in range(13):
                xx1 = jnp.maximum(bx1, x1r[pl.ds(q * 16, 16)])
                yy1 = jnp.maximum(by1, y1r[pl.ds(q * 16, 16)])
                xx2 = jnp.minimum(bx2, x2r[pl.ds(q * 16, 16)])
                yy2 = jnp.minimum(by2, y2r[pl.ds(q * 16, 16)])
                inter = (jnp.maximum(xx2 - xx1, jnp.float32(0.0)) *
                         jnp.maximum(yy2 - yy1, jnp.float32(0.0)))
                a2 = arr[pl.ds(q * 16, 16)]
                den = jnp.maximum(ba + a2 - inter, jnp.float32(1e-8))
                iou = inter / den
                supp = jnp.logical_and(iou > jnp.float32(_IOU_T), valid)
                s[q] = jnp.where(supp, _NEG_INF, s[q])

            @pl.when(valid)
            def _():
                _sstore(st_s, i, best)
                boxv = jnp.where(lanes == 0, bx1,
                                 jnp.where(lanes == 1, by1,
                                           jnp.where(lanes == 2, bx2, by2)))
                plsc.store_scatter(st_b, [4 * i + lanes], boxv,
                                   mask=lanes < 4)

            return (i + 1, jnp.logical_not(valid), tuple(s))

        with jax.named_scope("nms"):
            lax.while_loop(nms_cond, nms_body,
                           (jnp.int32(0), False, tuple(svecs)))

        pltpu.sync_copy(st_s.at[pl.ds(0, 104)],
                        ssc.at[pl.ds(lb * 8320 + c * 104, 104)])
        pltpu.sync_copy(st_b.at[pl.ds(0, 416)],
                        sbx.at[pl.ds(lb * 33280 + c * 416, 416)])
        return carry0

    lax.fori_loop(0, 10, one_problem, 0)
    plsc.subcore_barrier()

    @pl.when(sid < 2)
    def _():
        lb = sid
        bsel = 2 * cid + lb
        pltpu.sync_copy(ssc.at[pl.ds(lb * 8320, 8320)], msc)
        pltpu.sync_copy(sbx.at[pl.ds(lb * 33280, 33280)], mbx)

        def key_body(t, kcarry):
            for u in range(4):
                off = t * 64 + u * 16
                s = msc[pl.ds(off, 16)]
                bbits = plsc.bitcast(s, jnp.int32)
                mk[pl.ds(off, 16)] = jnp.where(s > jnp.float32(0.0),
                                               bbits, zk16)
            return kcarry

        lax.fori_loop(0, 8320 // 64, key_body, 0)
        for q in range(5):
            cvec = lanes + (q * 16)
            hk[pl.ds(q * 16, 16)] = plsc.load_gather(mk, [cvec * 104])
            hp[pl.ds(q * 16, 16)] = jnp.zeros((16,), jnp.int32)

        zf = jnp.zeros((16,), jnp.float32)
        for i in range(32):
            ob_st[pl.ds(i * 16, 16)] = zf
        for i in range(8):
            os_st[pl.ds(i * 16, 16)] = zf
            oc_st[pl.ds(i * 16, 16)] = zf
        onv_st[pl.ds(0, 16)] = jnp.zeros((16,), jnp.int32)

        def mg_cond(carry):
            jj, done, nv = carry
            return jnp.logical_and(jj < _MAXDET, jnp.logical_not(done))

        def mg_body(carry):
            jj, _, nv = carry
            mval = jnp.full((16,), _I32_MIN, jnp.int32)
            midx = jnp.zeros((16,), jnp.int32)
            for q in range(5):
                v = hk[pl.ds(q * 16, 16)]
                take = v > mval
                mval = jnp.where(take, v, mval)
                midx = jnp.where(take, lanes + (q * 16), midx)
            bestk = jnp.max(mval)
            valid = bestk > _I32_MIN
            cand = jnp.where(mval == bestk, midx, jnp.int32(1 << 30))
            bcls = jnp.where(valid, jnp.min(cand), jnp.int32(0))

            @pl.when(valid)
            def _():
                pos = _sload(hp, bcls)
                f = bcls * 104 + pos
                _sstore(os_st, jj, _sload(msc, f))
                _sstore(oc_st, jj, bcls.astype(jnp.float32))
                boxv = plsc.load_gather(mbx, [4 * f + lanes])
                plsc.store_scatter(ob_st, [4 * jj + lanes], boxv,
                                   mask=lanes < 4)
                _sstore(hp, bcls, pos + 1)
                _sstore(hk, bcls, _sload(mk, f + 1))

            return (jj + 1, jnp.logical_not(valid),
                    jnp.where(valid, nv + 1, nv))

        _, _, nv = lax.while_loop(mg_cond, mg_body,
                                  (jnp.int32(0), False, jnp.int32(0)))
        _sstore(onv_st, jnp.int32(0), nv)
        pltpu.sync_copy(ob_st, ob_hbm.at[pl.ds(bsel * 512, 512)])
        pltpu.sync_copy(os_st, os_hbm.at[pl.ds(bsel * 128, 128)])
        pltpu.sync_copy(oc_st, oc_hbm.at[pl.ds(bsel * 128, 128)])
        pltpu.sync_copy(onv_st.at[pl.ds(0, 8)],
                        onv_hbm.at[pl.ds(bsel * 8, 8)])


@functools.lru_cache(maxsize=2)
def _build_call(image_h, image_w):
    anch = _anchors_np(image_h, image_w)
    mesh = plsc.VectorSubcoreMesh(core_axis_name="c", subcore_axis_name="s")
    f32 = jnp.float32
    i32 = jnp.int32
    kern = pl.kernel(
        _sc_body,
        out_type=(jax.ShapeDtypeStruct((_BATCH * 512,), f32),
                  jax.ShapeDtypeStruct((_BATCH * 128,), f32),
                  jax.ShapeDtypeStruct((_BATCH * 128,), f32),
                  jax.ShapeDtypeStruct((_BATCH * 8,), i32)),
        mesh=mesh,
        compiler_params=pltpu.CompilerParams(needs_layout_passes=False,
                                             use_tc_tiling_on_sc=False),
        scratch_types=[
            pltpu.VMEM((_NPAD,), f32),
            pltpu.VMEM((_BUF,), i32), pltpu.VMEM((_BUF,), i32),
            pltpu.VMEM((_BUF,), i32),
            pltpu.VMEM((112,), i32), pltpu.VMEM((112,), i32),
            pltpu.VMEM((224, 16), f32),
            pltpu.VMEM((208,), f32), pltpu.VMEM((208,), f32),
            pltpu.VMEM((208,), f32), pltpu.VMEM((208,), f32),
            pltpu.VMEM((208,), f32),
            pltpu.VMEM((112,), f32), pltpu.VMEM((448,), f32),
            pltpu.VMEM((8320,), f32), pltpu.VMEM((33280,), f32),
            pltpu.VMEM((8320,), i32),
            pltpu.VMEM((80,), i32), pltpu.VMEM((80,), i32),
            pltpu.VMEM((512,), f32), pltpu.VMEM((128,), f32),
            pltpu.VMEM((128,), f32), pltpu.VMEM((16,), i32),
            pltpu.SemaphoreType.DMA,
            pltpu.VMEM_SHARED((2 * 8320,), f32),
            pltpu.VMEM_SHARED((2 * 33280,), f32),
        ],
    )

    def run(predictions):
        lg = jnp.transpose(predictions[:, :, 4:], (0, 2, 1))
        lg = lg.reshape(_BATCH * _NUM_CLASSES * _N)
        boxp = predictions[:, :, :4].reshape(_BATCH * _N, 4)
        anch_t = jnp.broadcast_to(jnp.asarray(anch)[None],
                                  (_BATCH, _N, 4))
        anch_t = anch_t.reshape(_BATCH * _N, 4)
        pad = jnp.zeros((_BATCH * _N, 8), jnp.float32)
        tab = jnp.concatenate([boxp, anch_t, pad], axis=1)
        ob, osc, ocl, onv = kern(lg, tab)
        return (ob.reshape(_BATCH, 128, 4)[:, :_MAXDET],
                osc.reshape(_BATCH, 128)[:, :_MAXDET],
                ocl.reshape(_BATCH, 128)[:, :_MAXDET],
                onv.reshape(_BATCH, 8)[:, 0])

    return run


def kernel(images, predictions):
    run = _build_call(images.shape[1], images.shape[2])
    return run(predictions)


# speedup vs baseline: 46.5284x; 1.4112x over previous
import functools

import numpy as np
import jax
import jax.numpy as jnp
from jax import lax
from jax.experimental import pallas as pl
from jax.experimental.pallas import tpu as pltpu
from jax.experimental.pallas import tpu_sc as plsc

_NUM_CLASSES = 80
_BATCH = 4
_N = 49104
_NPAD = 49152
_TOPK = 200
_MAXDET = 100
_IOU_T = 0.5
_SCORE_T = 0.05
_BUF = 672
_SHRINK_AT = 496
_CHUNK = 16368
_GRP = 11
_I32_MIN = -2147483648
_I32_MAX = 2147483647
_NEG_INF = float("-inf")


def _anchors_np(image_h, image_w):
    aspect_ratios = [0.5, 1.0, 2.0]
    scales = [2.0 ** x for x in [0.0, 1.0 / 3.0, 2.0 / 3.0]]
    areas = [x ** 2 for x in [32.0, 64.0, 128.0, 256.0, 512.0]]
    all_anchors = []
    for level in range(3, 8):
        stride = 2 ** level
        fh = int(np.ceil(image_h / stride))
        fw = int(np.ceil(image_w / stride))
        dims = []
        area = areas[level - 3]
        for ratio in aspect_ratios:
            ah = np.sqrt(area / ratio)
            aw = area / ah
            for scale in scales:
                dims.append([scale * aw, scale * ah])
        dims = np.array(dims, dtype=np.float32)
        rx = (np.arange(fw, dtype=np.float32) + 0.5) * stride
        ry = (np.arange(fh, dtype=np.float32) + 0.5) * stride
        cx, cy = np.meshgrid(rx, ry)
        centers = np.stack([cx, cy], axis=-1)
        centers = np.tile(centers[:, :, None, :], [1, 1, 9, 1])
        d = np.tile(dims[None, None, :, :], [fh, fw, 1, 1])
        a = np.concatenate([centers, d], axis=-1).reshape(-1, 4)
        all_anchors.append(a.astype(np.float32))
    return np.concatenate(all_anchors, axis=0)


def _lane():
    return lax.iota(jnp.int32, 16)


def _splat_gather(ref, idx):
    return plsc.load_gather(ref, [jnp.zeros((16,), jnp.int32) + idx])


def _sload(ref, idx):
    return _splat_gather(ref, idx)[0]


def _sstore(ref, idx, val):
    iv = jnp.zeros((16,), jnp.int32) + idx
    vv = jnp.zeros((16,), val.dtype) + val
    plsc.store_scatter(ref, [iv], vv, mask=_lane() == 0)


def _popcnt(mask):
    return plsc.all_reduce_population_count(mask)


def _to_key(v):
    b = plsc.bitcast(v, jnp.int32)
    return jnp.where(b < 0, b ^ jnp.int32(0x7FFFFFFF), b)


def _from_key(k):
    b = jnp.where(k < 0, k ^ jnp.int32(0x7FFFFFFF), k)
    return plsc.bitcast(b, jnp.float32)


def _fill_keys(iref, lgbuf, kref):
    for i in range(_BUF // 16):
        iv = iref[pl.ds(i * 16, 16)]
        v = plsc.load_gather(lgbuf, [iv])
        kref[pl.ds(i * 16, 16)] = _to_key(v)


def _rank_thresh(kref, r):
    def bit_body(_, lohi):
        lo, hi = lohi
        d = hi - lo
        half = lax.shift_right_logical(d, 1)
        mid = lo + half + (d & 1)
        midv = jnp.full((16,), 0, jnp.int32) + mid
        acc = jnp.zeros((16,), jnp.int32)
        for i in range(_BUF // 16):
            acc = acc + _popcnt(kref[pl.ds(i * 16, 16)] >= midv)
        ge = acc[0] >= r
        lo = jnp.where(ge, mid, lo)
        hi = jnp.where(ge, hi, mid - 1)
        return lo, hi
    lo, _ = lax.fori_loop(0, 32, bit_body,
                          (jnp.int32(_I32_MIN), jnp.int32(_I32_MAX)))
    return lo


def _count_gt(kref, t):
    tv = jnp.full((16,), 0, jnp.int32) + t
    acc = jnp.zeros((16,), jnp.int32)
    for i in range(_BUF // 16):
        acc = acc + _popcnt(kref[pl.ds(i * 16, 16)] > tv)
    return acc[0]


def _compact(kref, siref, diref, theta, tie_budget):
    pad = jnp.full((16,), _N, jnp.int32)
    for i in range(_BUF // 16):
        diref[pl.ds(i * 16, 16)] = pad
    tv = jnp.full((16,), 0, jnp.int32) + theta
    bv = jnp.full((16,), 0, jnp.int32) + tie_budget
    p = jnp.int32(0)
    ties = jnp.zeros((16,), jnp.int32)
    for i in range(_BUF // 16):
        k = kref[pl.ds(i * 16, 16)]
        iv = siref[pl.ds(i * 16, 16)]
        gt = k > tv
        eq = k == tv
        eqc = plsc.cumsum(eq.astype(jnp.int32))
        keep = jnp.logical_or(gt, jnp.logical_and(eq, (ties + eqc) <= bv))
        plsc.store_compressed(diref.at[pl.ds(p, 16)], iv, mask=keep)
        p = p + _popcnt(keep)[0]
        ties = ties + _popcnt(eq)
    return p


def _sc_body(logits_hbm, tab_hbm,
             ob_hbm, os_hbm, oc_hbm, onv_hbm,
             lgbuf, biA, biB, bkS, idxg0, idxg1, prows,
             x1r, y1r, x2r, y2r, arr, st_s, st_b,
             msc, mbx, mk, hk, hp, ob_st, os_st, oc_st, onv_st,
             dsem, ssc, sbx):
    cid = lax.axis_index("c")
    sid = lax.axis_index("s")
    lanes = _lane()
    zk16 = jnp.full((16,), _I32_MIN, jnp.int32)
    nanv = plsc.bitcast(jnp.full((16,), -1, jnp.int32), jnp.float32)

    def _row_of(p):
        lb = p // 5
        j = p - 5 * lb
        b = 2 * cid + lb
        c = sid * 5 + j
        return b, c, lb, b * _NUM_CLASSES + c

    def _issue_chunks(r):
        for ch in range(3):
            pltpu.async_copy(
                logits_hbm.at[pl.ds(r * _N + ch * _CHUNK, _CHUNK)],
                lgbuf.at[pl.ds(ch * _CHUNK, _CHUNK)], dsem)

    def one_problem(p, carry0):
        b, c, lb, r = _row_of(p)

        for i in range(3):
            lgbuf[pl.ds(_N + i * 16, 16)] = nanv
        for i in range(_BUF // 16):
            biA[pl.ds(i * 16, 16)] = jnp.full((16,), _N, jnp.int32)

        def chunk_body(ch, carry):
            P, theta = carry
            pltpu.make_async_copy(
                logits_hbm.at[pl.ds(0, _CHUNK)],
                lgbuf.at[pl.ds(0, _CHUNK)], dsem).wait()
            cbase = ch * _CHUNK

            def scan_body(t, carry2):
                P2, th2 = carry2
                thv = jnp.full((16,), 0, jnp.int32) + th2
                base = cbase + t * (_GRP * 16)
                ms = []
                pcs = []
                for u in range(_GRP):
                    k = _to_key(lgbuf[pl.ds(base + u * 16, 16)])
                    m = k > thv
                    ms.append(m)
                    pcs.append(_popcnt(m)[0])
                offs = [P2]
                for u in range(_GRP):
                    offs.append(offs[u] + pcs[u])
                for u in range(_GRP):
                    plsc.store_compressed(biA.at[pl.ds(offs[u], 16)],
                                          lanes + (base + u * 16),
                                          mask=ms[u])
                P2 = offs[_GRP]

                def do_shrink(op):
                    _fill_keys(biA, lgbuf, bkS)
                    t200 = _rank_thresh(bkS, _TOPK)
                    newP = _compact(bkS, biA, biB, t200, jnp.int32(_TOPK))
                    for i in range(_BUF // 16):
                        biA[pl.ds(i * 16, 16)] = biB[pl.ds(i * 16, 16)]
                    return newP, t200

                P2, th2 = lax.cond(P2 >= _SHRINK_AT, do_shrink,
                                   lambda op: op, (P2, th2))
                return P2, th2

            return lax.fori_loop(0, _CHUNK // (_GRP * 16), scan_body,
                                 (P, theta))

        lax.fori_loop(0, 3, chunk_body,
                      (jnp.int32(0), jnp.int32(_I32_MIN)))

        if True:
            _fill_keys(biA, lgbuf, bkS)
            tstar = _rank_thresh(bkS, _TOPK)
            m_gt = _count_gt(bkS, tstar)
            _compact(bkS, biA, biB, tstar, _TOPK - m_gt)

        bpv = jnp.full((16,), 0, jnp.int32) + b * _N
        for i in range(7):
            idxg0[pl.ds(i * 16, 16)] = biB[pl.ds(i * 16, 16)] + bpv
        for i in range(7):
            idxg1[pl.ds(i * 16, 16)] = biB[pl.ds((i + 7) * 16, 16)] + bpv
        if True:
            pltpu.sync_copy(tab_hbm.at[idxg0], prows.at[pl.ds(0, 112), :])
            pltpu.sync_copy(tab_hbm.at[idxg1],
                            prows.at[pl.ds(112, 112), :])

        svecs = []
        for i in range(13):
            rows = lanes + (i * 16)
            c0 = jnp.zeros((16,), jnp.int32)
            px = plsc.load_gather(prows, [rows, c0])
            py = plsc.load_gather(prows, [rows, c0 + 1])
            pw = plsc.load_gather(prows, [rows, c0 + 2])
            ph = plsc.load_gather(prows, [rows, c0 + 3])
            ax = plsc.load_gather(prows, [rows, c0 + 4])
            ay = plsc.load_gather(prows, [rows, c0 + 5])
            aw = plsc.load_gather(prows, [rows, c0 + 6])
            ah = plsc.load_gather(prows, [rows, c0 + 7])
            x = px * jnp.float32(0.1) * aw + ax
            y = py * jnp.float32(0.1) * ah + ay
            w = jnp.exp(pw * jnp.float32(0.2)) * aw
            h = jnp.exp(ph * jnp.float32(0.2)) * ah
            x1 = x - w * jnp.float32(0.5)
            y1 = y - h * jnp.float32(0.5)
            x2 = x + w * jnp.float32(0.5)
            y2 = y + h * jnp.float32(0.5)
            x1r[pl.ds(i * 16, 16)] = x1
            y1r[pl.ds(i * 16, 16)] = y1
            x2r[pl.ds(i * 16, 16)] = x2
            y2r[pl.ds(i * 16, 16)] = y2
            arr[pl.ds(i * 16, 16)] = (x2 - x1) * (y2 - y1)
            iv = biB[pl.ds(i * 16, 16)]
            lg = plsc.load_gather(lgbuf, [iv])
            s0 = jnp.float32(1.0) / (jnp.float32(1.0) + jnp.exp(-lg))
            ok = s0 >= jnp.float32(_SCORE_T)
            if i == 12:
                ok = jnp.logical_and(ok, lanes < 8)
            svecs.append(jnp.where(ok, s0, _NEG_INF))

        @pl.when(p < 9)
        def _():
            _issue_chunks(_row_of(p + 1)[3])

        zf = jnp.zeros((16,), jnp.float32)
        for i in range(7):
            st_s[pl.ds(i * 16, 16)] = zf
        for i in range(28):
            st_b[pl.ds(i * 16, 16)] = zf

        def nms_cond(carry):
            i, done = carry[0], carry[1]
            return jnp.logical_and(i < _MAXDET, jnp.logical_not(done))

        def nms_body(carry):
            i = carry[0]
            s = list(carry[2])
            mval = jnp.full((16,), _NEG_INF, jnp.float32)
            midx = jnp.zeros((16,), jnp.int32)
            for q in range(13):
                take = s[q] > mval
                mval = jnp.where(take, s[q], mval)
                midx = jnp.where(take, lanes + (q * 16), midx)
            best = jnp.max(mval)
            valid = best > jnp.float32(0.0)
            cand = jnp.where(mval == best, midx, jnp.int32(1 << 30))
            bidx = jnp.where(valid, jnp.min(cand), jnp.int32(0))

            bx1 = _splat_gather(x1r, bidx)
            by1 = _splat_gather(y1r, bidx)
            bx2 = _splat_gather(x2r, bidx)
            by2 = _splat_gather(y2r, bidx)
            ba = _splat_gather(arr, bidx)
            for q in range(13):
                xx1 = jnp.maximum(bx1, x1r[pl.ds(q * 16, 16)])
                yy1 = jnp.maximum(by1, y1r[pl.ds(q * 16, 16)])
                xx2 = jnp.minimum(bx2, x2r[pl.ds(q * 16, 16)])
                yy2 = jnp.minimum(by2, y2r[pl.ds(q * 16, 16)])
                inter = (jnp.maximum(xx2 - xx1, jnp.float32(0.0)) *
                         jnp.maximum(yy2 - yy1, jnp.float32(0.0)))
                a2 = arr[pl.ds(q * 16, 16)]
                den = jnp.maximum(ba + a2 - inter, jnp.float32(1e-8))
                iou = inter / den
                supp = jnp.logical_and(iou > jnp.float32(_IOU_T), valid)
                s[q] = jnp.where(supp, _NEG_INF, s[q])

            @pl.when(valid)
            def _():
                _sstore(st_s, i, best)
                boxv = jnp.where(lanes == 0, bx1,
                                 jnp.where(lanes == 1, by1,
                                           jnp.where(lanes == 2, bx2, by2)))
                plsc.store_scatter(st_b, [4 * i + lanes], boxv,
                                   mask=lanes < 4)

            return (i + 1, jnp.logical_not(valid), tuple(s))

        if True:
            lax.while_loop(nms_cond, nms_body,
                           (jnp.int32(0), False, tuple(svecs)))

        pltpu.sync_copy(st_s.at[pl.ds(0, 104)],
                        ssc.at[pl.ds(lb * 8320 + c * 104, 104)])
        pltpu.sync_copy(st_b.at[pl.ds(0, 416)],
                        sbx.at[pl.ds(lb * 33280 + c * 416, 416)])
        return carry0

    _issue_chunks(_row_of(jnp.int32(0))[3])
    lax.fori_loop(0, 10, one_problem, 0)
    plsc.subcore_barrier()

    @pl.when(sid < 2)
    def _():
        lb = sid
        bsel = 2 * cid + lb
        pltpu.sync_copy(ssc.at[pl.ds(lb * 8320, 8320)], msc)
        pltpu.sync_copy(sbx.at[pl.ds(lb * 33280, 33280)], mbx)

        def key_body(t, kcarry):
            for u in range(4):
                off = t * 64 + u * 16
                s = msc[pl.ds(off, 16)]
                bbits = plsc.bitcast(s, jnp.int32)
                mk[pl.ds(off, 16)] = jnp.where(s > jnp.float32(0.0),
                                               bbits, zk16)
            return kcarry

        lax.fori_loop(0, 8320 // 64, key_body, 0)
        for q in range(5):
            cvec = lanes + (q * 16)
            hk[pl.ds(q * 16, 16)] = plsc.load_gather(mk, [cvec * 104])
            hp[pl.ds(q * 16, 16)] = jnp.zeros((16,), jnp.int32)

        zf = jnp.zeros((16,), jnp.float32)
        for i in range(32):
            ob_st[pl.ds(i * 16, 16)] = zf
        for i in range(8):
            os_st[pl.ds(i * 16, 16)] = zf
            oc_st[pl.ds(i * 16, 16)] = zf
        onv_st[pl.ds(0, 16)] = jnp.zeros((16,), jnp.int32)

        def mg_cond(carry):
            jj, done, nv = carry
            return jnp.logical_and(jj < _MAXDET, jnp.logical_not(done))

        def mg_body(carry):
            jj, _, nv = carry
            mval = jnp.full((16,), _I32_MIN, jnp.int32)
            midx = jnp.zeros((16,), jnp.int32)
            for q in range(5):
                v = hk[pl.ds(q * 16, 16)]
                take = v > mval
                mval = jnp.where(take, v, mval)
                midx = jnp.where(take, lanes + (q * 16), midx)
            bestk = jnp.max(mval)
            valid = bestk > _I32_MIN
            cand = jnp.where(mval == bestk, midx, jnp.int32(1 << 30))
            bcls = jnp.where(valid, jnp.min(cand), jnp.int32(0))

            @pl.when(valid)
            def _():
                pos = _sload(hp, bcls)
                f = bcls * 104 + pos
                _sstore(os_st, jj, _sload(msc, f))
                _sstore(oc_st, jj, bcls.astype(jnp.float32))
                boxv = plsc.load_gather(mbx, [4 * f + lanes])
                plsc.store_scatter(ob_st, [4 * jj + lanes], boxv,
                                   mask=lanes < 4)
                _sstore(hp, bcls, pos + 1)
                _sstore(hk, bcls, _sload(mk, f + 1))

            return (jj + 1, jnp.logical_not(valid),
                    jnp.where(valid, nv + 1, nv))

        _, _, nv = lax.while_loop(mg_cond, mg_body,
                                  (jnp.int32(0), False, jnp.int32(0)))
        _sstore(onv_st, jnp.int32(0), nv)
        pltpu.sync_copy(ob_st, ob_hbm.at[pl.ds(bsel * 512, 512)])
        pltpu.sync_copy(os_st, os_hbm.at[pl.ds(bsel * 128, 128)])
        pltpu.sync_copy(oc_st, oc_hbm.at[pl.ds(bsel * 128, 128)])
        pltpu.sync_copy(onv_st.at[pl.ds(0, 8)],
                        onv_hbm.at[pl.ds(bsel * 8, 8)])


@functools.lru_cache(maxsize=2)
def _build_call(image_h, image_w):
    anch = _anchors_np(image_h, image_w)
    mesh = plsc.VectorSubcoreMesh(core_axis_name="c", subcore_axis_name="s")
    f32 = jnp.float32
    i32 = jnp.int32
    kern = pl.kernel(
        _sc_body,
        out_type=(jax.ShapeDtypeStruct((_BATCH * 512,), f32),
                  jax.ShapeDtypeStruct((_BATCH * 128,), f32),
                  jax.ShapeDtypeStruct((_BATCH * 128,), f32),
                  jax.ShapeDtypeStruct((_BATCH * 8,), i32)),
        mesh=mesh,
        compiler_params=pltpu.CompilerParams(needs_layout_passes=False,
                                             use_tc_tiling_on_sc=False),
        scratch_types=[
            pltpu.VMEM((_NPAD,), f32),
            pltpu.VMEM((_BUF,), i32), pltpu.VMEM((_BUF,), i32),
            pltpu.VMEM((_BUF,), i32),
            pltpu.VMEM((112,), i32), pltpu.VMEM((112,), i32),
            pltpu.VMEM((224, 16), f32),
            pltpu.VMEM((208,), f32), pltpu.VMEM((208,), f32),
            pltpu.VMEM((208,), f32), pltpu.VMEM((208,), f32),
            pltpu.VMEM((208,), f32),
            pltpu.VMEM((112,), f32), pltpu.VMEM((448,), f32),
            pltpu.VMEM((8320,), f32), pltpu.VMEM((33280,), f32),
            pltpu.VMEM((8320,), i32),
            pltpu.VMEM((80,), i32), pltpu.VMEM((80,), i32),
            pltpu.VMEM((512,), f32), pltpu.VMEM((128,), f32),
            pltpu.VMEM((128,), f32), pltpu.VMEM((16,), i32),
            pltpu.SemaphoreType.DMA,
            pltpu.VMEM_SHARED((2 * 8320,), f32),
            pltpu.VMEM_SHARED((2 * 33280,), f32),
        ],
    )

    def run(predictions):
        lg = jnp.transpose(predictions[:, :, 4:], (0, 2, 1))
        lg = lg.reshape(_BATCH * _NUM_CLASSES * _N)
        boxp = predictions[:, :, :4].reshape(_BATCH * _N, 4)
        anch_t = jnp.broadcast_to(jnp.asarray(anch)[None],
                                  (_BATCH, _N, 4))
        anch_t = anch_t.reshape(_BATCH * _N, 4)
        pad = jnp.zeros((_BATCH * _N, 8), jnp.float32)
        tab = jnp.concatenate([boxp, anch_t, pad], axis=1)
        ob, osc, ocl, onv = kern(lg, tab)
        return (ob.reshape(_BATCH, 128, 4)[:, :_MAXDET],
                osc.reshape(_BATCH, 128)[:, :_MAXDET],
                ocl.reshape(_BATCH, 128)[:, :_MAXDET],
                onv.reshape(_BATCH, 8)[:, 0])

    return run


def kernel(images, predictions):
    run = _build_call(images.shape[1], images.shape[2])
    return run(predictions)


# speedup vs baseline: 47.1111x; 1.0125x over previous
import functools

import numpy as np
import jax
import jax.numpy as jnp
from jax import lax
from jax.experimental import pallas as pl
from jax.experimental.pallas import tpu as pltpu
from jax.experimental.pallas import tpu_sc as plsc

_NUM_CLASSES = 80
_BATCH = 4
_N = 49104
_NPAD = 49152
_TOPK = 200
_MAXDET = 100
_IOU_T = 0.5
_SCORE_T = 0.05
_BUF = 672
_SHRINK_AT = 496
_CHUNK = 16368
_GRP = 11
_I32_MIN = -2147483648
_I32_MAX = 2147483647
_NEG_INF = float("-inf")


def _anchors_np(image_h, image_w):
    aspect_ratios = [0.5, 1.0, 2.0]
    scales = [2.0 ** x for x in [0.0, 1.0 / 3.0, 2.0 / 3.0]]
    areas = [x ** 2 for x in [32.0, 64.0, 128.0, 256.0, 512.0]]
    all_anchors = []
    for level in range(3, 8):
        stride = 2 ** level
        fh = int(np.ceil(image_h / stride))
        fw = int(np.ceil(image_w / stride))
        dims = []
        area = areas[level - 3]
        for ratio in aspect_ratios:
            ah = np.sqrt(area / ratio)
            aw = area / ah
            for scale in scales:
                dims.append([scale * aw, scale * ah])
        dims = np.array(dims, dtype=np.float32)
        rx = (np.arange(fw, dtype=np.float32) + 0.5) * stride
        ry = (np.arange(fh, dtype=np.float32) + 0.5) * stride
        cx, cy = np.meshgrid(rx, ry)
        centers = np.stack([cx, cy], axis=-1)
        centers = np.tile(centers[:, :, None, :], [1, 1, 9, 1])
        d = np.tile(dims[None, None, :, :], [fh, fw, 1, 1])
        a = np.concatenate([centers, d], axis=-1).reshape(-1, 4)
        all_anchors.append(a.astype(np.float32))
    return np.concatenate(all_anchors, axis=0)


def _lane():
    return lax.iota(jnp.int32, 16)


def _splat_gather(ref, idx):
    return plsc.load_gather(ref, [jnp.zeros((16,), jnp.int32) + idx])


def _sload(ref, idx):
    return _splat_gather(ref, idx)[0]


def _sstore(ref, idx, val):
    iv = jnp.zeros((16,), jnp.int32) + idx
    vv = jnp.zeros((16,), val.dtype) + val
    plsc.store_scatter(ref, [iv], vv, mask=_lane() == 0)


def _popcnt(mask):
    return plsc.all_reduce_population_count(mask)


def _to_key(v):
    b = plsc.bitcast(v, jnp.int32)
    return jnp.where(b < 0, b ^ jnp.int32(0x7FFFFFFF), b)


def _from_key(k):
    b = jnp.where(k < 0, k ^ jnp.int32(0x7FFFFFFF), k)
    return plsc.bitcast(b, jnp.float32)


def _fill_keys(iref, lgbuf, kref):
    for i in range(_BUF // 16):
        iv = iref[pl.ds(i * 16, 16)]
        v = plsc.load_gather(lgbuf, [iv])
        kref[pl.ds(i * 16, 16)] = _to_key(v)


def _rank_thresh(kref, r):
    def bit_body(_, lohi):
        lo, hi = lohi
        d = hi - lo
        half = lax.shift_right_logical(d, 1)
        mid = lo + half + (d & 1)
        midv = jnp.full((16,), 0, jnp.int32) + mid
        acc = jnp.zeros((16,), jnp.int32)
        for i in range(_BUF // 16):
            acc = acc + _popcnt(kref[pl.ds(i * 16, 16)] >= midv)
        ge = acc[0] >= r
        lo = jnp.where(ge, mid, lo)
        hi = jnp.where(ge, hi, mid - 1)
        return lo, hi
    lo, _ = lax.fori_loop(0, 32, bit_body,
                          (jnp.int32(_I32_MIN), jnp.int32(_I32_MAX)))
    return lo


def _count_gt(kref, t):
    tv = jnp.full((16,), 0, jnp.int32) + t
    acc = jnp.zeros((16,), jnp.int32)
    for i in range(_BUF // 16):
        acc = acc + _popcnt(kref[pl.ds(i * 16, 16)] > tv)
    return acc[0]


def _compact(kref, siref, diref, theta, tie_budget):
    pad = jnp.full((16,), _N, jnp.int32)
    for i in range(_BUF // 16):
        diref[pl.ds(i * 16, 16)] = pad
    tv = jnp.full((16,), 0, jnp.int32) + theta
    bv = jnp.full((16,), 0, jnp.int32) + tie_budget
    p = jnp.int32(0)
    ties = jnp.zeros((16,), jnp.int32)
    for i in range(_BUF // 16):
        k = kref[pl.ds(i * 16, 16)]
        iv = siref[pl.ds(i * 16, 16)]
        gt = k > tv
        eq = k == tv
        eqc = plsc.cumsum(eq.astype(jnp.int32))
        keep = jnp.logical_or(gt, jnp.logical_and(eq, (ties + eqc) <= bv))
        plsc.store_compressed(diref.at[pl.ds(p, 16)], iv, mask=keep)
        p = p + _popcnt(keep)[0]
        ties = ties + _popcnt(eq)
    return p


def _sc_body(logits_hbm, tab_hbm,
             ob_hbm, os_hbm, oc_hbm, onv_hbm,
             lgbuf, biA, biB, bkS, idxg0, idxg1, prows,
             x1r, y1r, x2r, y2r, arr, st_s, st_b,
             msc, mbx, mk, hk, hp, ob_st, os_st, oc_st, onv_st,
             dsem, ssc, sbx):
    cid = lax.axis_index("c")
    sid = lax.axis_index("s")
    lanes = _lane()
    zk16 = jnp.full((16,), _I32_MIN, jnp.int32)
    nanv = plsc.bitcast(jnp.full((16,), -1, jnp.int32), jnp.float32)

    def _row_of(p):
        lb = p // 5
        j = p - 5 * lb
        b = 2 * cid + lb
        c = sid * 5 + j
        return b, c, lb, b * _NUM_CLASSES + c

    def _issue_chunks(r):
        for ch in range(3):
            pltpu.async_copy(
                logits_hbm.at[pl.ds(r * _N + ch * _CHUNK, _CHUNK)],
                lgbuf.at[pl.ds(ch * _CHUNK, _CHUNK)], dsem)

    def one_problem(p, carry0):
        b, c, lb, r = _row_of(p)

        for i in range(3):
            lgbuf[pl.ds(_N + i * 16, 16)] = nanv
        for i in range(_BUF // 16):
            biA[pl.ds(i * 16, 16)] = jnp.full((16,), _N, jnp.int32)

        def chunk_body(ch, carry):
            P, theta = carry
            pltpu.make_async_copy(
                logits_hbm.at[pl.ds(0, _CHUNK)],
                lgbuf.at[pl.ds(0, _CHUNK)], dsem).wait()
            cbase = ch * _CHUNK

            def scan_body(t, carry2):
                P2, th2 = carry2
                thv = jnp.full((16,), 0, jnp.int32) + th2
                base = cbase + t * (_GRP * 16)
                ms = []
                pcs = []
                for u in range(_GRP):
                    k = _to_key(lgbuf[pl.ds(base + u * 16, 16)])
                    m = k > thv
                    ms.append(m)
                    pcs.append(_popcnt(m)[0])
                offs = [P2]
                for u in range(_GRP):
                    offs.append(offs[u] + pcs[u])
                for u in range(_GRP):
                    plsc.store_compressed(biA.at[pl.ds(offs[u], 16)],
                                          lanes + (base + u * 16),
                                          mask=ms[u])
                P2 = offs[_GRP]

                def do_shrink(op):
                    _fill_keys(biA, lgbuf, bkS)
                    t200 = _rank_thresh(bkS, _TOPK)
                    newP = _compact(bkS, biA, biB, t200, jnp.int32(_TOPK))
                    for i in range(_BUF // 16):
                        biA[pl.ds(i * 16, 16)] = biB[pl.ds(i * 16, 16)]
                    return newP, t200

                P2, th2 = lax.cond(P2 >= _SHRINK_AT, do_shrink,
                                   lambda op: op, (P2, th2))
                return P2, th2

            return lax.fori_loop(0, _CHUNK // (_GRP * 16), scan_body,
                                 (P, theta))

        lax.fori_loop(0, 3, chunk_body,
                      (jnp.int32(0), jnp.int32(_I32_MIN)))

        if True:
            _fill_keys(biA, lgbuf, bkS)
            tstar = _rank_thresh(bkS, _TOPK)
            m_gt = _count_gt(bkS, tstar)
            _compact(bkS, biA, biB, tstar, _TOPK - m_gt)

        bpv = jnp.full((16,), 0, jnp.int32) + b * _N
        for i in range(7):
            idxg0[pl.ds(i * 16, 16)] = biB[pl.ds(i * 16, 16)] + bpv
        for i in range(7):
            idxg1[pl.ds(i * 16, 16)] = biB[pl.ds((i + 7) * 16, 16)] + bpv
        if True:
            pltpu.sync_copy(tab_hbm.at[idxg0], prows.at[pl.ds(0, 112), :])
            pltpu.sync_copy(tab_hbm.at[idxg1],
                            prows.at[pl.ds(112, 112), :])

        svecs = []
        for i in range(13):
            rows = lanes + (i * 16)
            c0 = jnp.zeros((16,), jnp.int32)
            px = plsc.load_gather(prows, [rows, c0])
            py = plsc.load_gather(prows, [rows, c0 + 1])
            pw = plsc.load_gather(prows, [rows, c0 + 2])
            ph = plsc.load_gather(prows, [rows, c0 + 3])
            ax = plsc.load_gather(prows, [rows, c0 + 4])
            ay = plsc.load_gather(prows, [rows, c0 + 5])
            aw = plsc.load_gather(prows, [rows, c0 + 6])
            ah = plsc.load_gather(prows, [rows, c0 + 7])
            x = px * jnp.float32(0.1) * aw + ax
            y = py * jnp.float32(0.1) * ah + ay
            w = jnp.exp(pw * jnp.float32(0.2)) * aw
            h = jnp.exp(ph * jnp.float32(0.2)) * ah
            x1 = x - w * jnp.float32(0.5)
            y1 = y - h * jnp.float32(0.5)
            x2 = x + w * jnp.float32(0.5)
            y2 = y + h * jnp.float32(0.5)
            x1r[pl.ds(i * 16, 16)] = x1
            y1r[pl.ds(i * 16, 16)] = y1
            x2r[pl.ds(i * 16, 16)] = x2
            y2r[pl.ds(i * 16, 16)] = y2
            arr[pl.ds(i * 16, 16)] = (x2 - x1) * (y2 - y1)
            iv = biB[pl.ds(i * 16, 16)]
            lg = plsc.load_gather(lgbuf, [iv])
            s0 = jnp.float32(1.0) / (jnp.float32(1.0) + jnp.exp(-lg))
            ok = s0 >= jnp.float32(_SCORE_T)
            if i == 12:
                ok = jnp.logical_and(ok, lanes < 8)
            svecs.append(jnp.where(ok, s0, _NEG_INF))

        @pl.when(p < 9)
        def _():
            _issue_chunks(_row_of(p + 1)[3])

        zf = jnp.zeros((16,), jnp.float32)
        for i in range(7):
            st_s[pl.ds(i * 16, 16)] = zf
        for i in range(28):
            st_b[pl.ds(i * 16, 16)] = zf

        def nms_cond(carry):
            i, done = carry[0], carry[1]
            return jnp.logical_and(i < _MAXDET, jnp.logical_not(done))

        def nms_body(carry):
            i = carry[0]
            s = list(carry[2])
            mval = jnp.full((16,), _NEG_INF, jnp.float32)
            midx = jnp.zeros((16,), jnp.int32)
            for q in range(13):
                take = s[q] > mval
                mval = jnp.where(take, s[q], mval)
                midx = jnp.where(take, lanes + (q * 16), midx)
            best = jnp.max(mval)
            valid = best > jnp.float32(0.0)
            cand = jnp.where(mval == best, midx, jnp.int32(1 << 30))
            bidx = jnp.where(valid, jnp.min(cand), jnp.int32(0))

            bx1 = _splat_gather(x1r, bidx)
            by1 = _splat_gather(y1r, bidx)
            bx2 = _splat_gather(x2r, bidx)
            by2 = _splat_gather(y2r, bidx)
            ba = _splat_gather(arr, bidx)
            for q in range(13):
                xx1 = jnp.maximum(bx1, x1r[pl.ds(q * 16, 16)])
                yy1 = jnp.maximum(by1, y1r[pl.ds(q * 16, 16)])
                xx2 = jnp.minimum(bx2, x2r[pl.ds(q * 16, 16)])
                yy2 = jnp.minimum(by2, y2r[pl.ds(q * 16, 16)])
                inter = (jnp.maximum(xx2 - xx1, jnp.float32(0.0)) *
                         jnp.maximum(yy2 - yy1, jnp.float32(0.0)))
                a2 = arr[pl.ds(q * 16, 16)]
                den = jnp.maximum(ba + a2 - inter, jnp.float32(1e-8))
                iou = inter / den
                supp = jnp.logical_and(iou > jnp.float32(_IOU_T), valid)
                s[q] = jnp.where(supp, _NEG_INF, s[q])

            @pl.when(valid)
            def _():
                _sstore(st_s, i, best)
                boxv = jnp.where(lanes == 0, bx1,
                                 jnp.where(lanes == 1, by1,
                                           jnp.where(lanes == 2, bx2, by2)))
                plsc.store_scatter(st_b, [4 * i + lanes], boxv,
                                   mask=lanes < 4)

            return (i + 1, jnp.logical_not(valid), tuple(s))

        if True:
            lax.while_loop(nms_cond, nms_body,
                           (jnp.int32(0), False, tuple(svecs)))

        pltpu.sync_copy(st_s.at[pl.ds(0, 104)],
                        ssc.at[pl.ds(lb * 8320 + c * 104, 104)])
        pltpu.sync_copy(st_b.at[pl.ds(0, 416)],
                        sbx.at[pl.ds(lb * 33280 + c * 416, 416)])
        return carry0

    _issue_chunks(_row_of(jnp.int32(0))[3])
    lax.fori_loop(0, 10, one_problem, 0)
    plsc.subcore_barrier()

    @pl.when(sid < 2)
    def _():
        lb = sid
        bsel = 2 * cid + lb
        pltpu.sync_copy(ssc.at[pl.ds(lb * 8320, 8320)], msc)
        pltpu.sync_copy(sbx.at[pl.ds(lb * 33280, 33280)], mbx)

        def key_body(t, kcarry):
            for u in range(4):
                off = t * 64 + u * 16
                s = msc[pl.ds(off, 16)]
                bbits = plsc.bitcast(s, jnp.int32)
                mk[pl.ds(off, 16)] = jnp.where(s > jnp.float32(0.0),
                                               bbits, zk16)
            return kcarry

        lax.fori_loop(0, 8320 // 64, key_body, 0)
        for q in range(5):
            cvec = lanes + (q * 16)
            hk[pl.ds(q * 16, 16)] = plsc.load_gather(mk, [cvec * 104])
            hp[pl.ds(q * 16, 16)] = jnp.zeros((16,), jnp.int32)

        zf = jnp.zeros((16,), jnp.float32)
        for i in range(32):
            ob_st[pl.ds(i * 16, 16)] = zf
        for i in range(8):
            os_st[pl.ds(i * 16, 16)] = zf
            oc_st[pl.ds(i * 16, 16)] = zf
        onv_st[pl.ds(0, 16)] = jnp.zeros((16,), jnp.int32)

        def mg_cond(carry):
            jj, done, nv = carry
            return jnp.logical_and(jj < _MAXDET, jnp.logical_not(done))

        def mg_body(carry):
            jj, _, nv = carry
            mval = jnp.full((16,), _I32_MIN, jnp.int32)
            midx = jnp.zeros((16,), jnp.int32)
            for q in range(5):
                v = hk[pl.ds(q * 16, 16)]
                take = v > mval
                mval = jnp.where(take, v, mval)
                midx = jnp.where(take, lanes + (q * 16), midx)
            bestk = jnp.max(mval)
            valid = bestk > _I32_MIN
            cand = jnp.where(mval == bestk, midx, jnp.int32(1 << 30))
            bcls = jnp.where(valid, jnp.min(cand), jnp.int32(0))

            @pl.when(valid)
            def _():
                pos = _sload(hp, bcls)
                f = bcls * 104 + pos
                _sstore(os_st, jj, _sload(msc, f))
                _sstore(oc_st, jj, bcls.astype(jnp.float32))
                boxv = plsc.load_gather(mbx, [4 * f + lanes])
                plsc.store_scatter(ob_st, [4 * jj + lanes], boxv,
                                   mask=lanes < 4)
                _sstore(hp, bcls, pos + 1)
                _sstore(hk, bcls, _sload(mk, f + 1))

            return (jj + 1, jnp.logical_not(valid),
                    jnp.where(valid, nv + 1, nv))

        _, _, nv = lax.while_loop(mg_cond, mg_body,
                                  (jnp.int32(0), False, jnp.int32(0)))
        _sstore(onv_st, jnp.int32(0), nv)
        pltpu.sync_copy(ob_st, ob_hbm.at[pl.ds(bsel * 512, 512)])
        pltpu.sync_copy(os_st, os_hbm.at[pl.ds(bsel * 128, 128)])
        pltpu.sync_copy(oc_st, oc_hbm.at[pl.ds(bsel * 128, 128)])
        pltpu.sync_copy(onv_st.at[pl.ds(0, 8)],
                        onv_hbm.at[pl.ds(bsel * 8, 8)])


@functools.lru_cache(maxsize=2)
def _build_call(image_h, image_w):
    anch = _anchors_np(image_h, image_w)
    mesh = plsc.VectorSubcoreMesh(core_axis_name="c", subcore_axis_name="s")
    f32 = jnp.float32
    i32 = jnp.int32
    kern = pl.kernel(
        _sc_body,
        out_type=(jax.ShapeDtypeStruct((_BATCH * 512,), f32),
                  jax.ShapeDtypeStruct((_BATCH * 128,), f32),
                  jax.ShapeDtypeStruct((_BATCH * 128,), f32),
                  jax.ShapeDtypeStruct((_BATCH * 8,), i32)),
        mesh=mesh,
        compiler_params=pltpu.CompilerParams(needs_layout_passes=False,
                                             use_tc_tiling_on_sc=False),
        scratch_types=[
            pltpu.VMEM((_NPAD,), f32),
            pltpu.VMEM((_BUF,), i32), pltpu.VMEM((_BUF,), i32),
            pltpu.VMEM((_BUF,), i32),
            pltpu.VMEM((112,), i32), pltpu.VMEM((112,), i32),
            pltpu.VMEM((224, 8), f32),
            pltpu.VMEM((208,), f32), pltpu.VMEM((208,), f32),
            pltpu.VMEM((208,), f32), pltpu.VMEM((208,), f32),
            pltpu.VMEM((208,), f32),
            pltpu.VMEM((112,), f32), pltpu.VMEM((448,), f32),
            pltpu.VMEM((8320,), f32), pltpu.VMEM((33280,), f32),
            pltpu.VMEM((8320,), i32),
            pltpu.VMEM((80,), i32), pltpu.VMEM((80,), i32),
            pltpu.VMEM((512,), f32), pltpu.VMEM((128,), f32),
            pltpu.VMEM((128,), f32), pltpu.VMEM((16,), i32),
            pltpu.SemaphoreType.DMA,
            pltpu.VMEM_SHARED((2 * 8320,), f32),
            pltpu.VMEM_SHARED((2 * 33280,), f32),
        ],
    )

    def run(predictions):
        lg = jnp.transpose(predictions[:, :, 4:], (0, 2, 1))
        lg = lg.reshape(_BATCH * _NUM_CLASSES * _N)
        boxp = predictions[:, :, :4].reshape(_BATCH * _N, 4)
        anch_t = jnp.broadcast_to(jnp.asarray(anch)[None],
                                  (_BATCH, _N, 4))
        anch_t = anch_t.reshape(_BATCH * _N, 4)
        tab = jnp.concatenate([boxp, anch_t], axis=1)
        ob, osc, ocl, onv = kern(lg, tab)
        return (ob.reshape(_BATCH, 128, 4)[:, :_MAXDET],
                osc.reshape(_BATCH, 128)[:, :_MAXDET],
                ocl.reshape(_BATCH, 128)[:, :_MAXDET],
                onv.reshape(_BATCH, 8)[:, 0])

    return run


def kernel(images, predictions):
    run = _build_call(images.shape[1], images.shape[2])
    return run(predictions)
